# speedup vs baseline: 1.2628x; 1.2628x over previous
import functools

import numpy as np
import jax
import jax.numpy as jnp
from jax.experimental import pallas as pl
from jax.experimental.pallas import tpu as pltpu

_K = 3
_GRID = 48
_SG = _GRID * _GRID
_CIN = 8


@functools.lru_cache(maxsize=None)
def _lane_shift():
    probe = np.arange(8 * 128, dtype=np.float32).reshape(8, 128)

    def _probe_kernel(x_ref, o_ref):
        o_ref[...] = pltpu.roll(x_ref[...], 5, 1)

    try:
        out = pl.pallas_call(
            _probe_kernel,
            out_shape=jax.ShapeDtypeStruct(probe.shape, jnp.float32),
        )(jnp.asarray(probe))
        out = np.asarray(jax.block_until_ready(out))
        if np.array_equal(out, np.roll(probe, 5, axis=1)):
            return lambda v, s: pltpu.roll(v, s, 1)
        if np.array_equal(out, np.roll(probe, -5, axis=1)):
            return lambda v, s: pltpu.roll(v, (-s) % v.shape[-1], 1)
    except Exception:
        pass
    return lambda v, s: jnp.roll(v, s, axis=-1)


def _fwd_kernel(x_ref, w1_ref, b1_ref, w2_ref, b2_ref, w3_ref, b3_ref,
                w4_ref, b4_ref, m0_ref, m1_ref, o_ref, *, nb, roll):
    L = nb * _SG

    def shift(v, d):
        s = (-d) % L
        return v if s == 0 else roll(v, s)

    def stack(v, offs):
        return jnp.concatenate([shift(v, d) for d in offs], axis=0)

    enc_offs = [kh * _GRID + kw for kh in range(_K) for kw in range(_K)]
    dec_offs = [(kh - (_K - 1)) * _GRID + (kw - (_K - 1))
                for kh in range(_K) for kw in range(_K)]

    x = jnp.concatenate([x_ref[n] for n in range(nb)], axis=-1)

    h = jnp.dot(w1_ref[...], stack(x, enc_offs),
                preferred_element_type=jnp.float32)
    h = jnp.maximum(h + b1_ref[...], 0.0).astype(jnp.bfloat16)

    h = jnp.dot(w2_ref[...], stack(h, enc_offs),
                preferred_element_type=jnp.float32)
    h = (jnp.maximum(h + b2_ref[...], 0.0) * m0_ref[...]).astype(jnp.bfloat16)

    h = jnp.dot(w3_ref[...], stack(h, dec_offs),
                preferred_element_type=jnp.float32)
    h = (jnp.maximum(h + b3_ref[...], 0.0) * m1_ref[...]).astype(jnp.bfloat16)

    h = jnp.dot(w4_ref[...], stack(h, dec_offs),
                preferred_element_type=jnp.float32) + b4_ref[...]

    for n in range(nb):
        o_ref[n] = h[:, n * _SG:(n + 1) * _SG]


def _enc_wstack(w):
    w = jnp.asarray(w, jnp.float32)
    co = w.shape[0]
    return jnp.transpose(w, (0, 2, 3, 1)).reshape(co, -1).astype(jnp.bfloat16)


def _dec_wstack(w):
    w = jnp.asarray(w, jnp.float32)[:, :, ::-1, ::-1]
    co = w.shape[1]
    return jnp.transpose(w, (1, 2, 3, 0)).reshape(co, -1).astype(jnp.bfloat16)


def _masks(nb):
    rows = np.arange(_GRID)[:, None]
    cols = np.arange(_GRID)[None, :]
    out = []
    for hv in (_GRID - 2 * (_K - 1), _GRID - (_K - 1)):
        m2d = ((rows < hv) & (cols < hv)).astype(np.float32)
        out.append(jnp.asarray(np.tile(m2d.reshape(-1), nb)[None, :]))
    return out


def kernel(x, x1, enc_w0, enc_b0, enc_w1, enc_b1,
           dec_w0, dec_b0, dec_w1, dec_b1):
    xx = jnp.concatenate((x, x1), axis=0).astype(jnp.bfloat16)
    N = xx.shape[0]
    xx = xx.reshape(N, _CIN, _SG)

    bb = 32
    nb = N // bb
    L = nb * _SG

    ws = [_enc_wstack(enc_w0), _enc_wstack(enc_w1),
          _dec_wstack(dec_w0), _dec_wstack(dec_w1)]
    bs = [jnp.asarray(b, jnp.float32).reshape(-1, 1)
          for b in (enc_b0, enc_b1, dec_b0, dec_b1)]
    m0, m1 = _masks(nb)

    operands = [xx]
    in_specs = [pl.BlockSpec((nb, _CIN, _SG), lambda b: (b, 0, 0))]
    for w, b in zip(ws, bs):
        operands += [w, b]
        in_specs += [pl.BlockSpec(w.shape, lambda b: (0, 0)),
                     pl.BlockSpec(b.shape, lambda b: (0, 0))]
    operands += [m0, m1]
    in_specs += [pl.BlockSpec((1, L), lambda b: (0, 0)),
                 pl.BlockSpec((1, L), lambda b: (0, 0))]

    cf = ws[-1].shape[0]
    fn = pl.pallas_call(
        functools.partial(_fwd_kernel, nb=nb, roll=_lane_shift()),
        out_shape=jax.ShapeDtypeStruct((N, cf, _SG), jnp.float32),
        grid_spec=pltpu.PrefetchScalarGridSpec(
            num_scalar_prefetch=0,
            grid=(bb,),
            in_specs=in_specs,
            out_specs=pl.BlockSpec((nb, cf, _SG), lambda b: (b, 0, 0))),
        compiler_params=pltpu.CompilerParams(
            dimension_semantics=("parallel",)),
    )
    y = fn(*operands)
    return y.reshape(N, cf, _GRID, _GRID)


# speedup vs baseline: 1.6776x; 1.3284x over previous
import functools

import numpy as np
import jax
import jax.numpy as jnp
from jax.experimental import pallas as pl
from jax.experimental.pallas import tpu as pltpu

_K = 3
_GRID = 48
_SG = _GRID * _GRID
_CIN = 8


@functools.lru_cache(maxsize=None)
def _lane_shift():
    probe = np.arange(8 * 128, dtype=np.float32).reshape(8, 128)

    def _probe_kernel(x_ref, o_ref):
        o_ref[...] = pltpu.roll(x_ref[...], 5, 1)

    try:
        out = pl.pallas_call(
            _probe_kernel,
            out_shape=jax.ShapeDtypeStruct(probe.shape, jnp.float32),
        )(jnp.asarray(probe))
        out = np.asarray(jax.block_until_ready(out))
        if np.array_equal(out, np.roll(probe, 5, axis=1)):
            return lambda v, s: pltpu.roll(v, s, 1)
        if np.array_equal(out, np.roll(probe, -5, axis=1)):
            return lambda v, s: pltpu.roll(v, (-s) % v.shape[-1], 1)
    except Exception:
        pass
    return lambda v, s: jnp.roll(v, s, axis=-1)


def _fwd_kernel(x_ref, w1_ref, b1_ref, w2_ref, b2_ref, w3_ref, b3_ref,
                w4_ref, b4_ref, m0_ref, m1_ref, o_ref, *, nb, roll):
    L = nb * _SG

    def shift(v, d):
        s = (-d) % L
        return v if s == 0 else roll(v, s)

    def stack(v, offs):
        return jnp.concatenate([shift(v, d) for d in offs], axis=0)

    enc_offs = [kh * _GRID + kw for kh in range(_K) for kw in range(_K)]

    x = jnp.concatenate([x_ref[n] for n in range(nb)], axis=-1)

    h = jnp.dot(w1_ref[...], stack(x, enc_offs),
                preferred_element_type=jnp.float32)
    h = jnp.maximum(h + b1_ref[...], 0.0).astype(jnp.bfloat16)

    h = jnp.dot(w2_ref[...], stack(h, enc_offs),
                preferred_element_type=jnp.float32)
    h = (jnp.maximum(h + b2_ref[...], 0.0) * m0_ref[...]).astype(jnp.bfloat16)

    def rc_layer(v, w_ref, b_ref):
        co3 = w_ref.shape[0]
        co = co3 // _K
        xs = jnp.concatenate([shift(v, kw - (_K - 1)) for kw in range(_K)],
                             axis=0)
        z = jnp.dot(w_ref[...], xs, preferred_element_type=jnp.float32)
        acc = b_ref[...]
        for kh in range(_K):
            zk = z[kh * co:(kh + 1) * co]
            acc = acc + shift(zk, (kh - (_K - 1)) * _GRID)
        return acc

    h = rc_layer(h, w3_ref, b3_ref)
    h = (jnp.maximum(h, 0.0) * m1_ref[...]).astype(jnp.bfloat16)

    h = rc_layer(h, w4_ref, b4_ref)

    for n in range(nb):
        o_ref[n] = h[:, n * _SG:(n + 1) * _SG]


def _enc_wstack(w):
    w = jnp.asarray(w, jnp.float32)
    co = w.shape[0]
    return jnp.transpose(w, (0, 2, 3, 1)).reshape(co, -1).astype(jnp.bfloat16)


def _dec_wstack(w):
    w = jnp.asarray(w, jnp.float32)[:, :, ::-1, ::-1]
    co = w.shape[1]
    return (jnp.transpose(w, (2, 1, 3, 0))
            .reshape(_K * co, -1).astype(jnp.bfloat16))


def _masks(nb):
    rows = np.arange(_GRID)[:, None]
    cols = np.arange(_GRID)[None, :]
    out = []
    for hv in (_GRID - 2 * (_K - 1), _GRID - (_K - 1)):
        m2d = ((rows < hv) & (cols < hv)).astype(np.float32)
        out.append(jnp.asarray(np.tile(m2d.reshape(-1), nb)[None, :]))
    return out


def kernel(x, x1, enc_w0, enc_b0, enc_w1, enc_b1,
           dec_w0, dec_b0, dec_w1, dec_b1):
    xx = jnp.concatenate((x, x1), axis=0).astype(jnp.bfloat16)
    N = xx.shape[0]
    xx = xx.reshape(N, _CIN, _SG)

    bb = 32
    nb = N // bb
    L = nb * _SG

    ws = [_enc_wstack(enc_w0), _enc_wstack(enc_w1),
          _dec_wstack(dec_w0), _dec_wstack(dec_w1)]
    bs = [jnp.asarray(b, jnp.float32).reshape(-1, 1)
          for b in (enc_b0, enc_b1, dec_b0, dec_b1)]
    m0, m1 = _masks(nb)

    operands = [xx]
    in_specs = [pl.BlockSpec((nb, _CIN, _SG), lambda b: (b, 0, 0))]
    for w, b in zip(ws, bs):
        operands += [w, b]
        in_specs += [pl.BlockSpec(w.shape, lambda b: (0, 0)),
                     pl.BlockSpec(b.shape, lambda b: (0, 0))]
    operands += [m0, m1]
    in_specs += [pl.BlockSpec((1, L), lambda b: (0, 0)),
                 pl.BlockSpec((1, L), lambda b: (0, 0))]

    cf = bs[-1].shape[0]
    fn = pl.pallas_call(
        functools.partial(_fwd_kernel, nb=nb, roll=_lane_shift()),
        out_shape=jax.ShapeDtypeStruct((N, cf, _SG), jnp.float32),
        grid_spec=pltpu.PrefetchScalarGridSpec(
            num_scalar_prefetch=0,
            grid=(bb,),
            in_specs=in_specs,
            out_specs=pl.BlockSpec((nb, cf, _SG), lambda b: (b, 0, 0))),
        compiler_params=pltpu.CompilerParams(
            dimension_semantics=("parallel",)),
    )
    y = fn(*operands)
    return y.reshape(N, cf, _GRID, _GRID)


# speedup vs baseline: 1.8635x; 1.1108x over previous
import functools

import numpy as np
import jax
import jax.numpy as jnp
from jax.experimental import pallas as pl
from jax.experimental.pallas import tpu as pltpu

_K = 3
_GRID = 48
_SG = _GRID * _GRID
_CIN = 8


@functools.lru_cache(maxsize=None)
def _lane_shift():
    probe = np.arange(8 * 128, dtype=np.float32).reshape(8, 128)

    def _probe_kernel(x_ref, o_ref):
        o_ref[...] = pltpu.roll(x_ref[...], 5, 1)

    try:
        out = pl.pallas_call(
            _probe_kernel,
            out_shape=jax.ShapeDtypeStruct(probe.shape, jnp.float32),
        )(jnp.asarray(probe))
        out = np.asarray(jax.block_until_ready(out))
        if np.array_equal(out, np.roll(probe, 5, axis=1)):
            return lambda v, s: pltpu.roll(v, s, 1)
        if np.array_equal(out, np.roll(probe, -5, axis=1)):
            return lambda v, s: pltpu.roll(v, (-s) % v.shape[-1], 1)
    except Exception:
        pass
    return lambda v, s: jnp.roll(v, s, axis=-1)


def _fwd_kernel(x_ref, w1_ref, b1_ref, w2_ref, b2_ref, w3_ref, b3_ref,
                w4_ref, b4_ref, m0_ref, m1_ref, o_ref, *, nb, nchains, roll):
    nb2 = nb // nchains
    L = nb2 * _SG

    def shift(v, d):
        s = (-d) % L
        return v if s == 0 else roll(v, s)

    def stack(v, offs):
        return jnp.concatenate([shift(v, d) for d in offs], axis=0)

    enc_offs = [kh * _GRID + kw for kh in range(_K) for kw in range(_K)]

    def rc_layer(v, w_ref, b_ref):
        co = w_ref.shape[0] // _K
        xs = jnp.concatenate([shift(v, kw - (_K - 1)) for kw in range(_K)],
                             axis=0)
        z = jnp.dot(w_ref[...], xs, preferred_element_type=jnp.float32)
        acc = b_ref[...]
        for kh in range(_K):
            zk = z[kh * co:(kh + 1) * co]
            acc = acc + shift(zk, (kh - (_K - 1)) * _GRID)
        return acc

    def enc1(v):
        h = jnp.dot(w1_ref[...], stack(v, enc_offs),
                    preferred_element_type=jnp.float32)
        return jnp.maximum(h + b1_ref[...], 0.0).astype(jnp.bfloat16)

    def enc2(v):
        h = jnp.dot(w2_ref[...], stack(v, enc_offs),
                    preferred_element_type=jnp.float32)
        return (jnp.maximum(h + b2_ref[...], 0.0)
                * m0_ref[...]).astype(jnp.bfloat16)

    def dec1(v):
        h = rc_layer(v, w3_ref, b3_ref)
        return (jnp.maximum(h, 0.0) * m1_ref[...]).astype(jnp.bfloat16)

    def dec2(v):
        return rc_layer(v, w4_ref, b4_ref)

    hs = [jnp.concatenate([x_ref[c * nb2 + n] for n in range(nb2)], axis=-1)
          for c in range(nchains)]
    for stage in (enc1, enc2, dec1, dec2):
        hs = [stage(h) for h in hs]
    for c in range(nchains):
        for n in range(nb2):
            o_ref[c * nb2 + n] = hs[c][:, n * _SG:(n + 1) * _SG]


def _enc_wstack(w):
    w = jnp.asarray(w, jnp.float32)
    co = w.shape[0]
    return jnp.transpose(w, (0, 2, 3, 1)).reshape(co, -1).astype(jnp.bfloat16)


def _dec_wstack(w):
    w = jnp.asarray(w, jnp.float32)[:, :, ::-1, ::-1]
    co = w.shape[1]
    return (jnp.transpose(w, (2, 1, 3, 0))
            .reshape(_K * co, -1).astype(jnp.bfloat16))


def _masks(nb):
    rows = np.arange(_GRID)[:, None]
    cols = np.arange(_GRID)[None, :]
    out = []
    for hv in (_GRID - 2 * (_K - 1), _GRID - (_K - 1)):
        m2d = ((rows < hv) & (cols < hv)).astype(np.float32)
        out.append(jnp.asarray(np.tile(m2d.reshape(-1), nb)[None, :]))
    return out


def kernel(x, x1, enc_w0, enc_b0, enc_w1, enc_b1,
           dec_w0, dec_b0, dec_w1, dec_b1):
    xx = jnp.concatenate((x, x1), axis=0).astype(jnp.bfloat16)
    N = xx.shape[0]
    xx = xx.reshape(N, _CIN, _SG)

    bb = 32
    nchains = 2
    nb = N // bb
    L = (nb // nchains) * _SG

    ws = [_enc_wstack(enc_w0), _enc_wstack(enc_w1),
          _dec_wstack(dec_w0), _dec_wstack(dec_w1)]
    bs = [jnp.asarray(b, jnp.float32).reshape(-1, 1)
          for b in (enc_b0, enc_b1, dec_b0, dec_b1)]
    m0, m1 = _masks(nb // nchains)

    operands = [xx]
    in_specs = [pl.BlockSpec((nb, _CIN, _SG), lambda b: (b, 0, 0))]
    for w, b in zip(ws, bs):
        operands += [w, b]
        in_specs += [pl.BlockSpec(w.shape, lambda b: (0, 0)),
                     pl.BlockSpec(b.shape, lambda b: (0, 0))]
    operands += [m0, m1]
    in_specs += [pl.BlockSpec((1, L), lambda b: (0, 0)),
                 pl.BlockSpec((1, L), lambda b: (0, 0))]

    cf = bs[-1].shape[0]
    fn = pl.pallas_call(
        functools.partial(_fwd_kernel, nb=nb, nchains=nchains,
                          roll=_lane_shift()),
        out_shape=jax.ShapeDtypeStruct((N, cf, _SG), jnp.float32),
        grid_spec=pltpu.PrefetchScalarGridSpec(
            num_scalar_prefetch=0,
            grid=(bb,),
            in_specs=in_specs,
            out_specs=pl.BlockSpec((nb, cf, _SG), lambda b: (b, 0, 0))),
        compiler_params=pltpu.CompilerParams(
            dimension_semantics=("parallel",)),
    )
    y = fn(*operands)
    return y.reshape(N, cf, _GRID, _GRID)


# speedup vs baseline: 1.8971x; 1.0180x over previous
import functools

import numpy as np
import jax
import jax.numpy as jnp
from jax.experimental import pallas as pl
from jax.experimental.pallas import tpu as pltpu

_K = 3
_GRID = 48
_SG = _GRID * _GRID
_CIN = 8


@functools.lru_cache(maxsize=None)
def _lane_shift():
    probe = np.arange(8 * 128, dtype=np.float32).reshape(8, 128)

    def _probe_kernel(x_ref, o_ref):
        o_ref[...] = pltpu.roll(x_ref[...], 5, 1)

    try:
        out = pl.pallas_call(
            _probe_kernel,
            out_shape=jax.ShapeDtypeStruct(probe.shape, jnp.float32),
        )(jnp.asarray(probe))
        out = np.asarray(jax.block_until_ready(out))
        if np.array_equal(out, np.roll(probe, 5, axis=1)):
            return lambda v, s: pltpu.roll(v, s, 1)
        if np.array_equal(out, np.roll(probe, -5, axis=1)):
            return lambda v, s: pltpu.roll(v, (-s) % v.shape[-1], 1)
    except Exception:
        pass
    return lambda v, s: jnp.roll(v, s, axis=-1)


def _fwd_kernel(x_ref, x1_ref, w1_ref, b1_ref, w2_ref, b2_ref, w3_ref, b3_ref,
                w4_ref, b4_ref, m0_ref, m1_ref, o_ref, *, nb, nchains, bb,
                roll):
    nb2 = nb // nchains
    L = nb2 * _SG

    def shift(v, d):
        s = (-d) % L
        return v if s == 0 else roll(v, s)

    def stack(v, offs):
        return jnp.concatenate([shift(v, d) for d in offs], axis=0)

    enc_offs = [kh * _GRID + kw for kh in range(_K) for kw in range(_K)]

    def rc_layer(v, w_ref, b_ref):
        co = w_ref.shape[0] // _K
        xs = jnp.concatenate([shift(v, kw - (_K - 1)) for kw in range(_K)],
                             axis=0)
        z = jnp.dot(w_ref[...], xs, preferred_element_type=jnp.float32)
        acc = b_ref[...]
        for kh in range(_K):
            zk = z[kh * co:(kh + 1) * co]
            acc = acc + shift(zk, (kh - (_K - 1)) * _GRID)
        return acc

    def enc1(v):
        h = jnp.dot(w1_ref[...], stack(v, enc_offs),
                    preferred_element_type=jnp.float32)
        return jnp.maximum(h + b1_ref[...], 0.0).astype(jnp.bfloat16)

    def enc2(v):
        h = jnp.dot(w2_ref[...], stack(v, enc_offs),
                    preferred_element_type=jnp.float32)
        return (jnp.maximum(h + b2_ref[...], 0.0)
                * m0_ref[...]).astype(jnp.bfloat16)

    def dec1(v):
        h = rc_layer(v, w3_ref, b3_ref)
        return (jnp.maximum(h, 0.0) * m1_ref[...]).astype(jnp.bfloat16)

    def dec2(v):
        return rc_layer(v, w4_ref, b4_ref)

    from_x = pl.program_id(0) < (bb // 2)

    def load_chain(c):
        xa = jnp.concatenate([x_ref[c * nb2 + n] for n in range(nb2)],
                             axis=-1)
        xb = jnp.concatenate([x1_ref[c * nb2 + n] for n in range(nb2)],
                             axis=-1)
        return jnp.where(from_x, xa, xb).astype(jnp.bfloat16)

    hs = [load_chain(c) for c in range(nchains)]
    for stage in (enc1, enc2, dec1, dec2):
        hs = [stage(h) for h in hs]
    for c in range(nchains):
        for n in range(nb2):
            o_ref[c * nb2 + n] = hs[c][:, n * _SG:(n + 1) * _SG]


def _enc_wstack(w):
    w = jnp.asarray(w, jnp.float32)
    co = w.shape[0]
    return jnp.transpose(w, (0, 2, 3, 1)).reshape(co, -1).astype(jnp.bfloat16)


def _dec_wstack(w):
    w = jnp.asarray(w, jnp.float32)[:, :, ::-1, ::-1]
    co = w.shape[1]
    return (jnp.transpose(w, (2, 1, 3, 0))
            .reshape(_K * co, -1).astype(jnp.bfloat16))


def _masks(nb):
    rows = np.arange(_GRID)[:, None]
    cols = np.arange(_GRID)[None, :]
    out = []
    for hv in (_GRID - 2 * (_K - 1), _GRID - (_K - 1)):
        m2d = ((rows < hv) & (cols < hv)).astype(np.float32)
        out.append(jnp.asarray(np.tile(m2d.reshape(-1), nb)[None, :]))
    return out


def kernel(x, x1, enc_w0, enc_b0, enc_w1, enc_b1,
           dec_w0, dec_b0, dec_w1, dec_b1):
    Nx = x.shape[0]
    N = 2 * Nx
    x = x.reshape(Nx, _CIN, _SG)
    x1 = x1.reshape(Nx, _CIN, _SG)

    bb = 32
    nchains = 2
    nb = N // bb
    L = (nb // nchains) * _SG
    nbx = bb // 2

    ws = [_enc_wstack(enc_w0), _enc_wstack(enc_w1),
          _dec_wstack(dec_w0), _dec_wstack(dec_w1)]
    bs = [jnp.asarray(b, jnp.float32).reshape(-1, 1)
          for b in (enc_b0, enc_b1, dec_b0, dec_b1)]
    m0, m1 = _masks(nb // nchains)

    operands = [x, x1]
    in_specs = [
        pl.BlockSpec((nb, _CIN, _SG),
                     lambda b: (jnp.minimum(b, nbx - 1), 0, 0)),
        pl.BlockSpec((nb, _CIN, _SG),
                     lambda b: (jnp.maximum(b - nbx, 0), 0, 0)),
    ]
    for w, b in zip(ws, bs):
        operands += [w, b]
        in_specs += [pl.BlockSpec(w.shape, lambda b: (0, 0)),
                     pl.BlockSpec(b.shape, lambda b: (0, 0))]
    operands += [m0, m1]
    in_specs += [pl.BlockSpec((1, L), lambda b: (0, 0)),
                 pl.BlockSpec((1, L), lambda b: (0, 0))]

    cf = bs[-1].shape[0]
    fn = pl.pallas_call(
        functools.partial(_fwd_kernel, nb=nb, nchains=nchains, bb=bb,
                          roll=_lane_shift()),
        out_shape=jax.ShapeDtypeStruct((N, cf, _SG), jnp.float32),
        grid_spec=pltpu.PrefetchScalarGridSpec(
            num_scalar_prefetch=0,
            grid=(bb,),
            in_specs=in_specs,
            out_specs=pl.BlockSpec((nb, cf, _SG), lambda b: (b, 0, 0))),
        compiler_params=pltpu.CompilerParams(
            dimension_semantics=("parallel",)),
    )
    y = fn(*operands)
    return y.reshape(N, cf, _GRID, _GRID)


# speedup vs baseline: 1.9896x; 1.0487x over previous
import functools

import numpy as np
import jax
import jax.numpy as jnp
from jax.experimental import pallas as pl
from jax.experimental.pallas import tpu as pltpu

_K = 3
_GRID = 48
_SG = _GRID * _GRID
_CIN = 8


@functools.lru_cache(maxsize=None)
def _lane_shift():
    probe = np.arange(8 * 128, dtype=np.float32).reshape(8, 128)

    def _probe_kernel(x_ref, o_ref):
        o_ref[...] = pltpu.roll(x_ref[...], 5, 1)

    try:
        out = pl.pallas_call(
            _probe_kernel,
            out_shape=jax.ShapeDtypeStruct(probe.shape, jnp.float32),
        )(jnp.asarray(probe))
        out = np.asarray(jax.block_until_ready(out))
        if np.array_equal(out, np.roll(probe, 5, axis=1)):
            return lambda v, s: pltpu.roll(v, s, 1)
        if np.array_equal(out, np.roll(probe, -5, axis=1)):
            return lambda v, s: pltpu.roll(v, (-s) % v.shape[-1], 1)
    except Exception:
        pass
    return lambda v, s: jnp.roll(v, s, axis=-1)


def _fwd_kernel(x_ref, x1_ref, w1_ref, b1_ref, w2_ref, b2_ref, w3_ref, b3_ref,
                w4_ref, b4_ref, m0_ref, m1_ref, o_ref, *, nb, nchains, bb,
                roll):
    nb2 = nb // nchains
    L = nb2 * _SG

    def shift(v, d):
        s = (-d) % L
        return v if s == 0 else roll(v, s)

    def stack(v, offs):
        return jnp.concatenate([shift(v, d) for d in offs], axis=0)

    enc_offs = [kh * _GRID + kw for kh in range(_K) for kw in range(_K)]

    def rc_layer(v, w_ref, b_ref, kw0, kh0):
        co = w_ref.shape[0] // _K
        xs = jnp.concatenate([shift(v, kw0 + kw) for kw in range(_K)],
                             axis=0)
        z = jnp.dot(w_ref[...], xs, preferred_element_type=jnp.float32)
        z = z.astype(jnp.bfloat16)
        acc = b_ref[...]
        for kh in range(_K):
            zk = z[kh * co:(kh + 1) * co]
            acc = acc + shift(zk, (kh0 + kh) * _GRID)
        return acc

    def enc1(v):
        h = jnp.dot(w1_ref[...], stack(v, enc_offs),
                    preferred_element_type=jnp.float32)
        return jnp.maximum(h + b1_ref[...], 0.0).astype(jnp.bfloat16)

    def enc2(v):
        h = jnp.dot(w2_ref[...], stack(v, enc_offs),
                    preferred_element_type=jnp.float32)
        return (jnp.maximum(h + b2_ref[...], 0.0)
                * m0_ref[...]).astype(jnp.bfloat16)

    def dec1(v):
        h = rc_layer(v, w3_ref, b3_ref, -(_K - 1), -(_K - 1))
        return (jnp.maximum(h, 0.0) * m1_ref[...]).astype(jnp.bfloat16)

    def dec2(v):
        return rc_layer(v, w4_ref, b4_ref, -(_K - 1), -(_K - 1))

    from_x = pl.program_id(0) < (bb // 2)

    def load_chain(c):
        xa = jnp.concatenate([x_ref[c * nb2 + n] for n in range(nb2)],
                             axis=-1)
        xb = jnp.concatenate([x1_ref[c * nb2 + n] for n in range(nb2)],
                             axis=-1)
        return jnp.where(from_x, xa, xb).astype(jnp.bfloat16)

    hs = [load_chain(c) for c in range(nchains)]
    for stage in (enc1, enc2, dec1, dec2):
        hs = [stage(h) for h in hs]
    for c in range(nchains):
        for n in range(nb2):
            o_ref[c * nb2 + n] = hs[c][:, n * _SG:(n + 1) * _SG]


def _enc_wstack(w):
    w = jnp.asarray(w, jnp.float32)
    co = w.shape[0]
    return jnp.transpose(w, (0, 2, 3, 1)).reshape(co, -1).astype(jnp.bfloat16)


def _rc_enc_wstack(w):
    w = jnp.asarray(w, jnp.float32)
    co = w.shape[0]
    return (jnp.transpose(w, (2, 0, 3, 1))
            .reshape(_K * co, -1).astype(jnp.bfloat16))


def _dec_wstack(w):
    w = jnp.asarray(w, jnp.float32)[:, :, ::-1, ::-1]
    co = w.shape[1]
    return (jnp.transpose(w, (2, 1, 3, 0))
            .reshape(_K * co, -1).astype(jnp.bfloat16))


def _masks(nb):
    rows = np.arange(_GRID)[:, None]
    cols = np.arange(_GRID)[None, :]
    out = []
    for hv in (_GRID - 2 * (_K - 1), _GRID - (_K - 1)):
        m2d = ((rows < hv) & (cols < hv)).astype(np.float32)
        out.append(jnp.asarray(np.tile(m2d.reshape(-1), nb)[None, :]))
    return out


def kernel(x, x1, enc_w0, enc_b0, enc_w1, enc_b1,
           dec_w0, dec_b0, dec_w1, dec_b1):
    Nx = x.shape[0]
    N = 2 * Nx
    x = x.reshape(Nx, _CIN, _SG)
    x1 = x1.reshape(Nx, _CIN, _SG)

    bb = 32
    nchains = 2
    nb = N // bb
    L = (nb // nchains) * _SG
    nbx = bb // 2

    ws = [_enc_wstack(enc_w0), _enc_wstack(enc_w1),
          _dec_wstack(dec_w0), _dec_wstack(dec_w1)]
    bs = [jnp.asarray(b, jnp.float32).reshape(-1, 1)
          for b in (enc_b0, enc_b1, dec_b0, dec_b1)]
    m0, m1 = _masks(nb // nchains)

    operands = [x, x1]
    in_specs = [
        pl.BlockSpec((nb, _CIN, _SG),
                     lambda b: (jnp.minimum(b, nbx - 1), 0, 0)),
        pl.BlockSpec((nb, _CIN, _SG),
                     lambda b: (jnp.maximum(b - nbx, 0), 0, 0)),
    ]
    for w, b in zip(ws, bs):
        operands += [w, b]
        in_specs += [pl.BlockSpec(w.shape, lambda b: (0, 0)),
                     pl.BlockSpec(b.shape, lambda b: (0, 0))]
    operands += [m0, m1]
    in_specs += [pl.BlockSpec((1, L), lambda b: (0, 0)),
                 pl.BlockSpec((1, L), lambda b: (0, 0))]

    cf = bs[-1].shape[0]
    fn = pl.pallas_call(
        functools.partial(_fwd_kernel, nb=nb, nchains=nchains, bb=bb,
                          roll=_lane_shift()),
        out_shape=jax.ShapeDtypeStruct((N, cf, _SG), jnp.float32),
        grid_spec=pltpu.PrefetchScalarGridSpec(
            num_scalar_prefetch=0,
            grid=(bb,),
            in_specs=in_specs,
            out_specs=pl.BlockSpec((nb, cf, _SG), lambda b: (b, 0, 0))),
        compiler_params=pltpu.CompilerParams(
            dimension_semantics=("parallel",)),
    )
    y = fn(*operands)
    return y.reshape(N, cf, _GRID, _GRID)


# speedup vs baseline: 2.0656x; 1.0382x over previous
import functools

import numpy as np
import jax
import jax.numpy as jnp
from jax.experimental import pallas as pl
from jax.experimental.pallas import tpu as pltpu

_K = 3
_GRID = 48
_SG = _GRID * _GRID
_CIN = 8


@functools.lru_cache(maxsize=None)
def _lane_shift():
    probe = np.arange(8 * 128, dtype=np.float32).reshape(8, 128)

    def _probe_kernel(x_ref, o_ref):
        o_ref[...] = pltpu.roll(x_ref[...], 5, 1)

    try:
        out = pl.pallas_call(
            _probe_kernel,
            out_shape=jax.ShapeDtypeStruct(probe.shape, jnp.float32),
        )(jnp.asarray(probe))
        out = np.asarray(jax.block_until_ready(out))
        if np.array_equal(out, np.roll(probe, 5, axis=1)):
            return lambda v, s: pltpu.roll(v, s, 1)
        if np.array_equal(out, np.roll(probe, -5, axis=1)):
            return lambda v, s: pltpu.roll(v, (-s) % v.shape[-1], 1)
    except Exception:
        pass
    return lambda v, s: jnp.roll(v, s, axis=-1)


def _fwd_kernel(x_ref, x1_ref, w1_ref, b1_ref, w2_ref, b2_ref, w3_ref, b3_ref,
                w4_ref, b4_ref, m0_ref, m1_ref, o_ref, *, nb, nchains, bb,
                roll):
    nb2 = nb // nchains
    L = nb2 * _SG

    def shift(v, d):
        s = (-d) % L
        return v if s == 0 else roll(v, s)

    def stack(v, offs):
        return jnp.concatenate([shift(v, d) for d in offs], axis=0)

    enc_offs = [kh * _GRID + kw for kh in range(_K) for kw in range(_K)]

    def rc_layer(v, w_ref, b_ref, kw0, kh0):
        co = w_ref.shape[0] // _K
        xs = jnp.concatenate([shift(v, kw0 + kw) for kw in range(_K)],
                             axis=0)
        z = jnp.dot(w_ref[...], xs, preferred_element_type=jnp.float32)
        z = z.astype(jnp.bfloat16)
        acc = b_ref[...]
        for kh in range(_K):
            zk = z[kh * co:(kh + 1) * co]
            acc = acc + shift(zk, (kh0 + kh) * _GRID)
        return acc

    def enc1(v):
        h = jnp.dot(w1_ref[...], stack(v, enc_offs),
                    preferred_element_type=jnp.float32)
        return jnp.maximum(h + b1_ref[...], 0.0).astype(jnp.bfloat16)

    def enc2(v):
        h = jnp.dot(w2_ref[...], stack(v, enc_offs),
                    preferred_element_type=jnp.float32)
        return (jnp.maximum(h + b2_ref[...], 0.0)
                * m0_ref[...]).astype(jnp.bfloat16)

    def dec1(v):
        h = rc_layer(v, w3_ref, b3_ref, -(_K - 1), -(_K - 1))
        return (jnp.maximum(h, 0.0) * m1_ref[...]).astype(jnp.bfloat16)

    def dec2(v):
        return rc_layer(v, w4_ref, b4_ref, -(_K - 1), -(_K - 1))

    from_x = pl.program_id(0) < (bb // 2)

    def load_chain(c):
        xa = jnp.concatenate([x_ref[c * nb2 + n] for n in range(nb2)],
                             axis=-1)
        xb = jnp.concatenate([x1_ref[c * nb2 + n] for n in range(nb2)],
                             axis=-1)
        return jnp.where(from_x, xa, xb).astype(jnp.bfloat16)

    hs = [load_chain(c) for c in range(nchains)]
    for stage in (enc1, enc2, dec1, dec2):
        hs = [stage(h) for h in hs]
    for c in range(nchains):
        for n in range(nb2):
            o_ref[c * nb2 + n] = hs[c][:, n * _SG:(n + 1) * _SG]


def _enc_wstack(w):
    w = jnp.asarray(w, jnp.float32)
    co = w.shape[0]
    return jnp.transpose(w, (0, 2, 3, 1)).reshape(co, -1).astype(jnp.bfloat16)


def _rc_enc_wstack(w):
    w = jnp.asarray(w, jnp.float32)
    co = w.shape[0]
    return (jnp.transpose(w, (2, 0, 3, 1))
            .reshape(_K * co, -1).astype(jnp.bfloat16))


def _dec_wstack(w):
    w = jnp.asarray(w, jnp.float32)[:, :, ::-1, ::-1]
    co = w.shape[1]
    return (jnp.transpose(w, (2, 1, 3, 0))
            .reshape(_K * co, -1).astype(jnp.bfloat16))


def _masks(nb):
    rows = np.arange(_GRID)[:, None]
    cols = np.arange(_GRID)[None, :]
    out = []
    for hv in (_GRID - 2 * (_K - 1), _GRID - (_K - 1)):
        m2d = ((rows < hv) & (cols < hv)).astype(np.float32)
        out.append(jnp.asarray(np.tile(m2d.reshape(-1), nb)[None, :]))
    return out


def kernel(x, x1, enc_w0, enc_b0, enc_w1, enc_b1,
           dec_w0, dec_b0, dec_w1, dec_b1):
    Nx = x.shape[0]
    N = 2 * Nx
    x = x.reshape(Nx, _CIN, _SG)
    x1 = x1.reshape(Nx, _CIN, _SG)

    bb = 16
    nchains = 2
    nb = N // bb
    L = (nb // nchains) * _SG
    nbx = bb // 2

    ws = [_enc_wstack(enc_w0), _enc_wstack(enc_w1),
          _dec_wstack(dec_w0), _dec_wstack(dec_w1)]
    bs = [jnp.asarray(b, jnp.float32).reshape(-1, 1)
          for b in (enc_b0, enc_b1, dec_b0, dec_b1)]
    m0, m1 = _masks(nb // nchains)

    operands = [x, x1]
    in_specs = [
        pl.BlockSpec((nb, _CIN, _SG),
                     lambda b: (jnp.minimum(b, nbx - 1), 0, 0)),
        pl.BlockSpec((nb, _CIN, _SG),
                     lambda b: (jnp.maximum(b - nbx, 0), 0, 0)),
    ]
    for w, b in zip(ws, bs):
        operands += [w, b]
        in_specs += [pl.BlockSpec(w.shape, lambda b: (0, 0)),
                     pl.BlockSpec(b.shape, lambda b: (0, 0))]
    operands += [m0, m1]
    in_specs += [pl.BlockSpec((1, L), lambda b: (0, 0)),
                 pl.BlockSpec((1, L), lambda b: (0, 0))]

    cf = bs[-1].shape[0]
    fn = pl.pallas_call(
        functools.partial(_fwd_kernel, nb=nb, nchains=nchains, bb=bb,
                          roll=_lane_shift()),
        out_shape=jax.ShapeDtypeStruct((N, cf, _SG), jnp.float32),
        grid_spec=pltpu.PrefetchScalarGridSpec(
            num_scalar_prefetch=0,
            grid=(bb,),
            in_specs=in_specs,
            out_specs=pl.BlockSpec((nb, cf, _SG), lambda b: (b, 0, 0))),
        compiler_params=pltpu.CompilerParams(
            dimension_semantics=("parallel",)),
    )
    y = fn(*operands)
    return y.reshape(N, cf, _GRID, _GRID)


# speedup vs baseline: 2.0660x; 1.0002x over previous
import functools

import numpy as np
import jax
import jax.numpy as jnp
from jax.experimental import pallas as pl
from jax.experimental.pallas import tpu as pltpu

_K = 3
_GRID = 48
_SG = _GRID * _GRID
_CIN = 8


@functools.lru_cache(maxsize=None)
def _lane_shift():
    probe = np.arange(8 * 128, dtype=np.float32).reshape(8, 128)

    def _probe_kernel(x_ref, o_ref):
        o_ref[...] = pltpu.roll(x_ref[...], 5, 1)

    try:
        out = pl.pallas_call(
            _probe_kernel,
            out_shape=jax.ShapeDtypeStruct(probe.shape, jnp.float32),
        )(jnp.asarray(probe))
        out = np.asarray(jax.block_until_ready(out))
        if np.array_equal(out, np.roll(probe, 5, axis=1)):
            return lambda v, s: pltpu.roll(v, s, 1)
        if np.array_equal(out, np.roll(probe, -5, axis=1)):
            return lambda v, s: pltpu.roll(v, (-s) % v.shape[-1], 1)
    except Exception:
        pass
    return lambda v, s: jnp.roll(v, s, axis=-1)


def _fwd_kernel(x_ref, x1_ref, w1_ref, b1_ref, w2_ref, b2_ref, w3_ref, b3_ref,
                w4_ref, b4_ref, m0_ref, m1_ref, o_ref, *, nb, nchains, bb,
                roll):
    nb2 = nb // nchains
    L = nb2 * _SG

    def shift(v, d):
        s = (-d) % L
        return v if s == 0 else roll(v, s)

    def stack(v, offs):
        return jnp.concatenate([shift(v, d) for d in offs], axis=0)

    enc_offs = [kh * _GRID + kw for kh in range(_K) for kw in range(_K)]

    def rc_layer(v, w_ref, b_ref, kw0, kh0):
        co = w_ref.shape[0] // _K
        xs = jnp.concatenate([shift(v, kw0 + kw) for kw in range(_K)],
                             axis=0)
        z = jnp.dot(w_ref[...], xs, preferred_element_type=jnp.float32)
        z = z.astype(jnp.bfloat16)
        acc = b_ref[...]
        for kh in range(_K):
            zk = z[kh * co:(kh + 1) * co]
            acc = acc + shift(zk, (kh0 + kh) * _GRID)
        return acc

    def enc1(v):
        h = jnp.dot(w1_ref[...], stack(v, enc_offs),
                    preferred_element_type=jnp.float32)
        return jnp.maximum(h + b1_ref[...], 0.0).astype(jnp.bfloat16)

    def enc2(v):
        h = jnp.dot(w2_ref[...], stack(v, enc_offs),
                    preferred_element_type=jnp.float32)
        return (jnp.maximum(h + b2_ref[...], 0.0)
                * m0_ref[...]).astype(jnp.bfloat16)

    def dec1(v):
        h = rc_layer(v, w3_ref, b3_ref, -(_K - 1), -(_K - 1))
        return (jnp.maximum(h, 0.0) * m1_ref[...]).astype(jnp.bfloat16)

    def dec2(v):
        return rc_layer(v, w4_ref, b4_ref, -(_K - 1), -(_K - 1))

    from_x = pl.program_id(0) < (bb // 2)

    def load_chain(c):
        xa = jnp.concatenate([x_ref[c * nb2 + n] for n in range(nb2)],
                             axis=-1)
        xb = jnp.concatenate([x1_ref[c * nb2 + n] for n in range(nb2)],
                             axis=-1)
        return jnp.where(from_x, xa, xb).astype(jnp.bfloat16)

    hs = [load_chain(c) for c in range(nchains)]
    for stage in (enc1, enc2, dec1, dec2):
        hs = [stage(h) for h in hs]
    for c in range(nchains):
        for n in range(nb2):
            o_ref[c * nb2 + n] = hs[c][:, n * _SG:(n + 1) * _SG]


def _enc_wstack(w):
    w = jnp.asarray(w, jnp.float32)
    co = w.shape[0]
    return jnp.transpose(w, (0, 2, 3, 1)).reshape(co, -1).astype(jnp.bfloat16)


def _dec_wstack(w):
    w = jnp.asarray(w, jnp.float32)[:, :, ::-1, ::-1]
    co = w.shape[1]
    return (jnp.transpose(w, (2, 1, 3, 0))
            .reshape(_K * co, -1).astype(jnp.bfloat16))


def _masks(nb):
    rows = np.arange(_GRID)[:, None]
    cols = np.arange(_GRID)[None, :]
    out = []
    for hv in (_GRID - 2 * (_K - 1), _GRID - (_K - 1)):
        m2d = ((rows < hv) & (cols < hv)).astype(np.float32)
        out.append(jnp.asarray(np.tile(m2d.reshape(-1), nb)[None, :]))
    return out


def kernel(x, x1, enc_w0, enc_b0, enc_w1, enc_b1,
           dec_w0, dec_b0, dec_w1, dec_b1):
    Nx = x.shape[0]
    N = 2 * Nx
    x = x.reshape(Nx, _CIN, _SG)
    x1 = x1.reshape(Nx, _CIN, _SG)

    bb = 16
    nchains = 2
    nb = N // bb
    L = (nb // nchains) * _SG
    nbx = bb // 2

    ws = [_enc_wstack(enc_w0), _enc_wstack(enc_w1),
          _dec_wstack(dec_w0), _dec_wstack(dec_w1)]
    bs = [jnp.asarray(b, jnp.float32).reshape(-1, 1)
          for b in (enc_b0, enc_b1, dec_b0, dec_b1)]
    m0, m1 = _masks(nb // nchains)

    operands = [x, x1]
    in_specs = [
        pl.BlockSpec((nb, _CIN, _SG),
                     lambda b: (jnp.minimum(b, nbx - 1), 0, 0)),
        pl.BlockSpec((nb, _CIN, _SG),
                     lambda b: (jnp.maximum(b - nbx, 0), 0, 0)),
    ]
    for w, b in zip(ws, bs):
        operands += [w, b]
        in_specs += [pl.BlockSpec(w.shape, lambda b: (0, 0)),
                     pl.BlockSpec(b.shape, lambda b: (0, 0))]
    operands += [m0, m1]
    in_specs += [pl.BlockSpec((1, L), lambda b: (0, 0)),
                 pl.BlockSpec((1, L), lambda b: (0, 0))]

    cf = bs[-1].shape[0]
    fn = pl.pallas_call(
        functools.partial(_fwd_kernel, nb=nb, nchains=nchains, bb=bb,
                          roll=_lane_shift()),
        out_shape=jax.ShapeDtypeStruct((N, cf, _SG), jnp.float32),
        grid_spec=pltpu.PrefetchScalarGridSpec(
            num_scalar_prefetch=0,
            grid=(bb,),
            in_specs=in_specs,
            out_specs=pl.BlockSpec((nb, cf, _SG), lambda b: (b, 0, 0))),
        compiler_params=pltpu.CompilerParams(
            dimension_semantics=("parallel",)),
    )
    y = fn(*operands)
    return y.reshape(N, cf, _GRID, _GRID)
